# baseline (speedup 1.0000x reference)
import functools

import jax
import jax.numpy as jnp
from jax import lax
from jax.experimental import pallas as pl
from jax.experimental.pallas import tpu as pltpu
from jax.experimental.pallas import tpu_sc as plsc

N = 10000
E = 320000
D = 128
NP = 10240

NC = 2
NS = 16
NW = NC * NS
EPW = E // NW
C = 80
NCH = EPW // C
RPS = NP // NS
RB = 1000
TB = 1024

PHASES = (32, 32, 32, 29)
PHB = PHASES[0]

_mesh = plsc.VectorSubcoreMesh(core_axis_name="c", subcore_axis_name="s")


@functools.partial(
    pl.kernel,
    out_type=jax.ShapeDtypeStruct((NC, 2, NP), jnp.float32),
    mesh=_mesh,
    scratch_types=[
        pltpu.VMEM((NCH, C), jnp.int32),
        pltpu.VMEM((NCH, C), jnp.int32),
        pltpu.VMEM((C,), jnp.float32),
        pltpu.VMEM_SHARED((NP,), jnp.float32),
        pltpu.VMEM_SHARED((NP,), jnp.float32),
        pltpu.SemaphoreType.DMA,
        pltpu.SemaphoreType.DMA,
    ],
)
def _sc_degrees(src_hbm, dst_hbm, zeros_hbm, ones_hbm, out_hbm,
                src_v, dst_v, ones_v, degs_sh, degd_sh, sem_s, sem_d):
    cid = lax.axis_index("c")
    sid = lax.axis_index("s")
    wid = cid * NS + sid

    pltpu.sync_copy(ones_hbm, ones_v)
    sl = pl.ds(sid * RPS, RPS)
    pltpu.sync_copy(zeros_hbm.at[sl], degs_sh.at[sl])
    pltpu.sync_copy(zeros_hbm.at[sl], degd_sh.at[sl])
    pltpu.sync_copy(src_hbm.at[wid], src_v)
    pltpu.sync_copy(dst_hbm.at[wid], dst_v)
    plsc.subcore_barrier()

    pltpu.async_copy(ones_v, degs_sh.at[src_v.at[0]], sem_s, add=True)
    pltpu.async_copy(ones_v, degd_sh.at[dst_v.at[0]], sem_d, add=True)

    @pl.loop(1, NCH)
    def _(j):
        pltpu.async_copy(ones_v, degs_sh.at[src_v.at[j]], sem_s, add=True)
        pltpu.async_copy(ones_v, degd_sh.at[dst_v.at[j]], sem_d, add=True)
        pltpu.make_async_copy(ones_v, degs_sh.at[src_v.at[j]], sem_s).wait()
        pltpu.make_async_copy(ones_v, degd_sh.at[dst_v.at[j]], sem_d).wait()

    pltpu.make_async_copy(ones_v, degs_sh.at[src_v.at[0]], sem_s).wait()
    pltpu.make_async_copy(ones_v, degd_sh.at[dst_v.at[0]], sem_d).wait()

    plsc.subcore_barrier()
    pltpu.sync_copy(degs_sh.at[sl], out_hbm.at[cid, 0, sl])
    pltpu.sync_copy(degd_sh.at[sl], out_hbm.at[cid, 1, sl])


def _ring3_phase(h_hbm, src_v, dst_v, bufs, sgs, sss, agg_sh, n):
    def g_wait(c, k):
        pltpu.make_async_copy(h_hbm.at[src_v.at[c]], bufs[k], sgs[k]).wait()

    def s_issue(c, k):
        pltpu.async_copy(bufs[k], agg_sh.at[dst_v.at[c]], sss[k], add=True)

    def s_wait(c, k):
        pltpu.make_async_copy(bufs[k], agg_sh.at[dst_v.at[c]], sss[k]).wait()

    def g_issue(c, k):
        pltpu.async_copy(h_hbm.at[src_v.at[c]], bufs[k], sgs[k])

    g_issue(0, 0)
    g_issue(1, 1)

    g_wait(0, 0)
    s_issue(0, 0)
    g_issue(2, 2)

    t_total = (n - 1) // 3

    @pl.loop(0, t_total)
    def _(t):
        c0 = 1 + 3 * t
        for p, k in ((0, 1), (1, 2), (2, 0)):
            c = c0 + p
            k2 = (k + 2) % 3
            g_wait(c, k)
            s_issue(c, k)
            s_wait(c - 1, k2)

            @pl.when(c + 2 < n)
            def _():
                g_issue(c + 2, k2)

    for c in range(1 + 3 * t_total, n):
        k = c % 3
        k2 = (k + 2) % 3
        g_wait(c, k)
        s_issue(c, k)
        s_wait(c - 1, k2)
        if c + 2 < n:
            g_issue(c + 2, k2)

    s_wait(n - 1, (n - 1) % 3)


@functools.partial(
    pl.kernel,
    out_type=jax.ShapeDtypeStruct((NC, N, D), jnp.float32),
    mesh=_mesh,
    scratch_types=[
        pltpu.VMEM((PHB, C), jnp.int32),
        pltpu.VMEM((PHB, C), jnp.int32),
        pltpu.VMEM((C, D), jnp.float32),
        pltpu.VMEM((C, D), jnp.float32),
        pltpu.VMEM((C, D), jnp.float32),
        pltpu.VMEM_SHARED((N, D), jnp.float32),
        pltpu.SemaphoreType.DMA,
        pltpu.SemaphoreType.DMA,
        pltpu.SemaphoreType.DMA,
        pltpu.SemaphoreType.DMA,
        pltpu.SemaphoreType.DMA,
        pltpu.SemaphoreType.DMA,
    ],
)
def _sc_aggregate(h_hbm, src_hbm, dst_hbm, zeros_hbm, out_hbm,
                  src_v, dst_v, buf0, buf1, buf2, agg_sh,
                  sg0, sg1, sg2, ss0, ss1, ss2):
    cid = lax.axis_index("c")
    sid = lax.axis_index("s")
    wid = cid * NS + sid
    bufs = (buf0, buf1, buf2)
    sgs = (sg0, sg1, sg2)
    sss = (ss0, ss1, ss2)

    @pl.when(sid < N // RB)
    def _():
        sl = pl.ds(pl.multiple_of(sid * RB, 8), RB)
        pltpu.sync_copy(zeros_hbm.at[sl], agg_sh.at[sl])

    base = 0
    pltpu.sync_copy(src_hbm.at[wid, pl.ds(0, PHASES[0])],
                    src_v.at[pl.ds(0, PHASES[0])])
    pltpu.sync_copy(dst_hbm.at[wid, pl.ds(0, PHASES[0])],
                    dst_v.at[pl.ds(0, PHASES[0])])
    plsc.subcore_barrier()

    for n in PHASES:
        _ring3_phase(h_hbm, src_v, dst_v, bufs, sgs, sss, agg_sh, n)
        base += n
        if base < NCH:
            nxt = min(PHB, NCH - base)
            pltpu.sync_copy(src_hbm.at[wid, pl.ds(base, nxt)],
                            src_v.at[pl.ds(0, nxt)])
            pltpu.sync_copy(dst_hbm.at[wid, pl.ds(base, nxt)],
                            dst_v.at[pl.ds(0, nxt)])

    plsc.subcore_barrier()

    @pl.when(sid < N // RB)
    def _():
        sl = pl.ds(pl.multiple_of(sid * RB, 8), RB)
        pltpu.sync_copy(agg_sh.at[sl], out_hbm.at[cid, sl])


def _tc_pre_body(feat_ref, w_ref, degp_ref, h_ref):
    d = degp_ref[...]
    deg = d[0, 0, 0, 0, :] + d[1, 0, 0, 0, :]
    norm = lax.rsqrt(jnp.maximum(deg, 1.0))
    t = jnp.dot(feat_ref[...], w_ref[...], preferred_element_type=jnp.float32)
    h_ref[...] = t * norm[:, None]


_tc_pre = pl.pallas_call(
    _tc_pre_body,
    grid=(NP // TB,),
    in_specs=[
        pl.BlockSpec((TB, D), lambda i: (i, 0)),
        pl.BlockSpec((D, D), lambda i: (0, 0)),
        pl.BlockSpec((NC, 2, 1, 1, TB), lambda i: (0, 0, i, 0, 0)),
    ],
    out_specs=pl.BlockSpec((TB, D), lambda i: (i, 0)),
    out_shape=jax.ShapeDtypeStruct((N, D), jnp.float32),
)


def _tc_post_body(parts_ref, degp_ref, b_ref, out_ref):
    p = parts_ref[...]
    d = degp_ref[...]
    deg = d[0, 1, 0, 0, :] + d[1, 1, 0, 0, :]
    norm = lax.rsqrt(jnp.maximum(deg, 1.0))
    agg = (p[0] + p[1]) * norm[:, None]
    out_ref[...] = jnp.maximum(agg + b_ref[...], 0.0)


_tc_post = pl.pallas_call(
    _tc_post_body,
    grid=(NP // TB,),
    in_specs=[
        pl.BlockSpec((NC, TB, D), lambda i: (0, i, 0)),
        pl.BlockSpec((NC, 2, 1, 1, TB), lambda i: (0, 0, i, 0, 0)),
        pl.BlockSpec((1, D), lambda i: (0, 0)),
    ],
    out_specs=pl.BlockSpec((TB, D), lambda i: (i, 0)),
    out_shape=jax.ShapeDtypeStruct((N, D), jnp.float32),
)


def kernel(feat, edge_index, W, b):
    src3 = edge_index[0].reshape(NW, NCH, C)
    dst3 = edge_index[1].reshape(NW, NCH, C)
    zeros1 = jnp.zeros((NP,), jnp.float32)
    ones_c = jnp.ones((C,), jnp.float32)
    zeros2 = jnp.zeros((N, D), jnp.float32)

    degp = _sc_degrees(src3, dst3, zeros1, ones_c)
    degp5 = degp.reshape(NC, 2, NP // TB, 1, TB)

    h = _tc_pre(feat, W, degp5)
    parts = _sc_aggregate(h, src3, dst3, zeros2)
    return _tc_post(parts, degp5, b.reshape(1, D))


# speedup vs baseline: 1.0814x; 1.0814x over previous
import functools

import jax
import jax.numpy as jnp
from jax import lax
from jax.experimental import pallas as pl
from jax.experimental.pallas import tpu as pltpu
from jax.experimental.pallas import tpu_sc as plsc

N = 10000
E = 320000
D = 128
NP = 10240

NC = 2
NS = 16
NW = NC * NS
EPW = E // NW
C = 80
NCH = EPW // C
RPS = NP // NS
RB = 1000
TB = 1024

PHASES = (32, 32, 32, 29)
PHB = PHASES[0]

_mesh = plsc.VectorSubcoreMesh(core_axis_name="c", subcore_axis_name="s")


@functools.partial(
    pl.kernel,
    out_type=jax.ShapeDtypeStruct((NC, 2, NP), jnp.float32),
    mesh=_mesh,
    scratch_types=[
        pltpu.VMEM((EPW,), jnp.int32),
        pltpu.VMEM((EPW,), jnp.int32),
        pltpu.VMEM((C,), jnp.float32),
        pltpu.VMEM_SHARED((NP,), jnp.float32),
        pltpu.VMEM_SHARED((NP,), jnp.float32),
        pltpu.SemaphoreType.DMA,
        pltpu.SemaphoreType.DMA,
    ],
)
def _sc_degrees(ei_hbm, zeros_hbm, ones_hbm, out_hbm,
                src_v, dst_v, ones_v, degs_sh, degd_sh, sem_s, sem_d):
    cid = lax.axis_index("c")
    sid = lax.axis_index("s")
    wid = cid * NS + sid

    pltpu.sync_copy(ones_hbm, ones_v)
    sl = pl.ds(sid * RPS, RPS)
    pltpu.sync_copy(zeros_hbm.at[sl], degs_sh.at[sl])
    pltpu.sync_copy(zeros_hbm.at[sl], degd_sh.at[sl])
    pltpu.sync_copy(ei_hbm.at[pl.ds(wid * EPW, EPW)], src_v)
    pltpu.sync_copy(ei_hbm.at[pl.ds(E + wid * EPW, EPW)], dst_v)
    plsc.subcore_barrier()

    def schunk(j):
        return src_v.at[pl.ds(j * C, C)]

    def dchunk(j):
        return dst_v.at[pl.ds(j * C, C)]

    pltpu.async_copy(ones_v, degs_sh.at[schunk(0)], sem_s, add=True)
    pltpu.async_copy(ones_v, degd_sh.at[dchunk(0)], sem_d, add=True)

    @pl.loop(1, NCH)
    def _(j):
        pltpu.async_copy(ones_v, degs_sh.at[schunk(j)], sem_s, add=True)
        pltpu.async_copy(ones_v, degd_sh.at[dchunk(j)], sem_d, add=True)
        pltpu.make_async_copy(ones_v, degs_sh.at[schunk(j)], sem_s).wait()
        pltpu.make_async_copy(ones_v, degd_sh.at[dchunk(j)], sem_d).wait()

    pltpu.make_async_copy(ones_v, degs_sh.at[schunk(0)], sem_s).wait()
    pltpu.make_async_copy(ones_v, degd_sh.at[dchunk(0)], sem_d).wait()

    plsc.subcore_barrier()
    pltpu.sync_copy(degs_sh.at[sl], out_hbm.at[cid, 0, sl])
    pltpu.sync_copy(degd_sh.at[sl], out_hbm.at[cid, 1, sl])


def _ring3_phase(h_hbm, src_v, dst_v, bufs, sgs, sss, agg_sh, n):
    def g_wait(c, k):
        pltpu.make_async_copy(
            h_hbm.at[src_v.at[pl.ds(c * C, C)]], bufs[k], sgs[k]).wait()

    def s_issue(c, k):
        pltpu.async_copy(
            bufs[k], agg_sh.at[dst_v.at[pl.ds(c * C, C)]], sss[k], add=True)

    def s_wait(c, k):
        pltpu.make_async_copy(
            bufs[k], agg_sh.at[dst_v.at[pl.ds(c * C, C)]], sss[k]).wait()

    def g_issue(c, k):
        pltpu.async_copy(h_hbm.at[src_v.at[pl.ds(c * C, C)]], bufs[k], sgs[k])

    g_issue(0, 0)
    g_issue(1, 1)

    g_wait(0, 0)
    s_issue(0, 0)
    g_issue(2, 2)

    t_total = (n - 1) // 3

    @pl.loop(0, t_total)
    def _(t):
        c0 = 1 + 3 * t
        for p, k in ((0, 1), (1, 2), (2, 0)):
            c = c0 + p
            k2 = (k + 2) % 3
            g_wait(c, k)
            s_issue(c, k)
            s_wait(c - 1, k2)

            @pl.when(c + 2 < n)
            def _():
                g_issue(c + 2, k2)

    for c in range(1 + 3 * t_total, n):
        k = c % 3
        k2 = (k + 2) % 3
        g_wait(c, k)
        s_issue(c, k)
        s_wait(c - 1, k2)
        if c + 2 < n:
            g_issue(c + 2, k2)

    s_wait(n - 1, (n - 1) % 3)


@functools.partial(
    pl.kernel,
    out_type=jax.ShapeDtypeStruct((NC, N, D), jnp.float32),
    mesh=_mesh,
    scratch_types=[
        pltpu.VMEM((PHB * C,), jnp.int32),
        pltpu.VMEM((PHB * C,), jnp.int32),
        pltpu.VMEM((C, D), jnp.float32),
        pltpu.VMEM((C, D), jnp.float32),
        pltpu.VMEM((C, D), jnp.float32),
        pltpu.VMEM_SHARED((N, D), jnp.float32),
        pltpu.SemaphoreType.DMA,
        pltpu.SemaphoreType.DMA,
        pltpu.SemaphoreType.DMA,
        pltpu.SemaphoreType.DMA,
        pltpu.SemaphoreType.DMA,
        pltpu.SemaphoreType.DMA,
    ],
)
def _sc_aggregate(h_hbm, ei_hbm, zeros_hbm, out_hbm,
                  src_v, dst_v, buf0, buf1, buf2, agg_sh,
                  sg0, sg1, sg2, ss0, ss1, ss2):
    cid = lax.axis_index("c")
    sid = lax.axis_index("s")
    wid = cid * NS + sid
    bufs = (buf0, buf1, buf2)
    sgs = (sg0, sg1, sg2)
    sss = (ss0, ss1, ss2)

    @pl.when(sid < N // RB)
    def _():
        sl = pl.ds(pl.multiple_of(sid * RB, 8), RB)
        pltpu.sync_copy(zeros_hbm.at[sl], agg_sh.at[sl])

    base = 0
    pltpu.sync_copy(ei_hbm.at[pl.ds(wid * EPW, PHASES[0] * C)],
                    src_v.at[pl.ds(0, PHASES[0] * C)])
    pltpu.sync_copy(ei_hbm.at[pl.ds(E + wid * EPW, PHASES[0] * C)],
                    dst_v.at[pl.ds(0, PHASES[0] * C)])
    plsc.subcore_barrier()

    for n in PHASES:
        _ring3_phase(h_hbm, src_v, dst_v, bufs, sgs, sss, agg_sh, n)
        base += n
        if base < NCH:
            nxt = min(PHB, NCH - base)
            pltpu.sync_copy(ei_hbm.at[pl.ds(wid * EPW + base * C, nxt * C)],
                            src_v.at[pl.ds(0, nxt * C)])
            pltpu.sync_copy(ei_hbm.at[pl.ds(E + wid * EPW + base * C, nxt * C)],
                            dst_v.at[pl.ds(0, nxt * C)])

    plsc.subcore_barrier()

    @pl.when(sid < N // RB)
    def _():
        sl = pl.ds(pl.multiple_of(sid * RB, 8), RB)
        pltpu.sync_copy(agg_sh.at[sl], out_hbm.at[cid, sl])


def _tc_pre_body(feat_ref, w_ref, degp_ref, h_ref):
    d = degp_ref[...]
    deg = d[0, 0, 0, 0, :] + d[1, 0, 0, 0, :]
    norm = lax.rsqrt(jnp.maximum(deg, 1.0))
    t = jnp.dot(feat_ref[...].astype(jnp.bfloat16),
                w_ref[...].astype(jnp.bfloat16),
                preferred_element_type=jnp.float32)
    h_ref[...] = t * norm[:, None]


_tc_pre = pl.pallas_call(
    _tc_pre_body,
    grid=(NP // TB,),
    in_specs=[
        pl.BlockSpec((TB, D), lambda i: (i, 0)),
        pl.BlockSpec((D, D), lambda i: (0, 0)),
        pl.BlockSpec((NC, 2, 1, 1, TB), lambda i: (0, 0, i, 0, 0)),
    ],
    out_specs=pl.BlockSpec((TB, D), lambda i: (i, 0)),
    out_shape=jax.ShapeDtypeStruct((N, D), jnp.float32),
)


def _tc_post_body(parts_ref, degp_ref, b_ref, out_ref):
    p = parts_ref[...]
    d = degp_ref[...]
    deg = d[0, 1, 0, 0, :] + d[1, 1, 0, 0, :]
    norm = lax.rsqrt(jnp.maximum(deg, 1.0))
    agg = (p[0] + p[1]) * norm[:, None]
    out_ref[...] = jnp.maximum(agg + b_ref[...], 0.0)


_tc_post = pl.pallas_call(
    _tc_post_body,
    grid=(NP // TB,),
    in_specs=[
        pl.BlockSpec((NC, TB, D), lambda i: (0, i, 0)),
        pl.BlockSpec((NC, 2, 1, 1, TB), lambda i: (0, 0, i, 0, 0)),
        pl.BlockSpec((1, D), lambda i: (0, 0)),
    ],
    out_specs=pl.BlockSpec((TB, D), lambda i: (i, 0)),
    out_shape=jax.ShapeDtypeStruct((N, D), jnp.float32),
)


def kernel(feat, edge_index, W, b):
    zeros1 = jnp.zeros((NP,), jnp.float32)
    ones_c = jnp.ones((C,), jnp.float32)
    zeros2 = jnp.zeros((N, D), jnp.float32)

    ei_flat = edge_index.reshape(2 * E)
    degp = _sc_degrees(ei_flat, zeros1, ones_c)
    degp5 = degp.reshape(NC, 2, NP // TB, 1, TB)

    h = _tc_pre(feat, W, degp5)
    parts = _sc_aggregate(h, ei_flat, zeros2)
    return _tc_post(parts, degp5, b.reshape(1, D))


# speedup vs baseline: 1.1280x; 1.0431x over previous
import functools

import jax
import jax.numpy as jnp
from jax import lax
from jax.experimental import pallas as pl
from jax.experimental.pallas import tpu as pltpu
from jax.experimental.pallas import tpu_sc as plsc

N = 10000
E = 320000
D = 128
NP = 10240

NC = 2
NS = 16
NW = NC * NS
EPW = E // NW
C = 80
NCH = EPW // C
RPS = NP // NS
RB = 1000
TB = 1024

PHASES = (63, 62)
PHB = PHASES[0]

_mesh = plsc.VectorSubcoreMesh(core_axis_name="c", subcore_axis_name="s")


@functools.partial(
    pl.kernel,
    out_type=jax.ShapeDtypeStruct((NC, 2, NP), jnp.float32),
    mesh=_mesh,
    scratch_types=[
        pltpu.VMEM((EPW,), jnp.int32),
        pltpu.VMEM((EPW,), jnp.int32),
        pltpu.VMEM((C,), jnp.float32),
        pltpu.VMEM_SHARED((NP,), jnp.float32),
        pltpu.VMEM_SHARED((NP,), jnp.float32),
        pltpu.SemaphoreType.DMA,
        pltpu.SemaphoreType.DMA,
    ],
)
def _sc_degrees(ei_hbm, zeros_hbm, ones_hbm, out_hbm,
                src_v, dst_v, ones_v, degs_sh, degd_sh, sem_s, sem_d):
    cid = lax.axis_index("c")
    sid = lax.axis_index("s")
    wid = cid * NS + sid

    pltpu.async_copy(ei_hbm.at[pl.ds(wid * EPW, EPW)], src_v, sem_s)
    pltpu.async_copy(ei_hbm.at[pl.ds(E + wid * EPW, EPW)], dst_v, sem_d)
    pltpu.sync_copy(ones_hbm, ones_v)
    sl = pl.ds(sid * RPS, RPS)
    pltpu.sync_copy(zeros_hbm.at[sl], degs_sh.at[sl])
    pltpu.sync_copy(zeros_hbm.at[sl], degd_sh.at[sl])
    pltpu.make_async_copy(ei_hbm.at[pl.ds(wid * EPW, EPW)], src_v, sem_s).wait()
    pltpu.make_async_copy(ei_hbm.at[pl.ds(E + wid * EPW, EPW)], dst_v,
                          sem_d).wait()
    plsc.subcore_barrier()

    def schunk(j):
        return src_v.at[pl.ds(j * C, C)]

    def dchunk(j):
        return dst_v.at[pl.ds(j * C, C)]

    pltpu.async_copy(ones_v, degs_sh.at[schunk(0)], sem_s, add=True)
    pltpu.async_copy(ones_v, degd_sh.at[dchunk(0)], sem_d, add=True)

    @pl.loop(1, NCH)
    def _(j):
        pltpu.async_copy(ones_v, degs_sh.at[schunk(j)], sem_s, add=True)
        pltpu.async_copy(ones_v, degd_sh.at[dchunk(j)], sem_d, add=True)
        pltpu.make_async_copy(ones_v, degs_sh.at[schunk(j)], sem_s).wait()
        pltpu.make_async_copy(ones_v, degd_sh.at[dchunk(j)], sem_d).wait()

    pltpu.make_async_copy(ones_v, degs_sh.at[schunk(0)], sem_s).wait()
    pltpu.make_async_copy(ones_v, degd_sh.at[dchunk(0)], sem_d).wait()

    plsc.subcore_barrier()
    pltpu.sync_copy(degs_sh.at[sl], out_hbm.at[cid, 0, sl])
    pltpu.sync_copy(degd_sh.at[sl], out_hbm.at[cid, 1, sl])


def _ring3_phase(h_hbm, src_v, dst_v, bufs, sgs, sss, agg_sh, n):
    def g_wait(c, k):
        pltpu.make_async_copy(
            h_hbm.at[src_v.at[pl.ds(c * C, C)]], bufs[k], sgs[k]).wait()

    def s_issue(c, k):
        pltpu.async_copy(
            bufs[k], agg_sh.at[dst_v.at[pl.ds(c * C, C)]], sss[k], add=True)

    def s_wait(c, k):
        pltpu.make_async_copy(
            bufs[k], agg_sh.at[dst_v.at[pl.ds(c * C, C)]], sss[k]).wait()

    def g_issue(c, k):
        pltpu.async_copy(h_hbm.at[src_v.at[pl.ds(c * C, C)]], bufs[k], sgs[k])

    g_issue(0, 0)
    g_issue(1, 1)

    g_wait(0, 0)
    s_issue(0, 0)
    g_issue(2, 2)

    t_total = (n - 1) // 3

    @pl.loop(0, t_total)
    def _(t):
        c0 = 1 + 3 * t
        for p, k in ((0, 1), (1, 2), (2, 0)):
            c = c0 + p
            k2 = (k + 2) % 3
            g_wait(c, k)
            s_issue(c, k)
            s_wait(c - 1, k2)

            @pl.when(c + 2 < n)
            def _():
                g_issue(c + 2, k2)

    for c in range(1 + 3 * t_total, n):
        k = c % 3
        k2 = (k + 2) % 3
        g_wait(c, k)
        s_issue(c, k)
        s_wait(c - 1, k2)
        if c + 2 < n:
            g_issue(c + 2, k2)

    s_wait(n - 1, (n - 1) % 3)


@functools.partial(
    pl.kernel,
    out_type=jax.ShapeDtypeStruct((NC, N, D), jnp.float32),
    mesh=_mesh,
    scratch_types=[
        pltpu.VMEM((PHB * C,), jnp.int32),
        pltpu.VMEM((PHB * C,), jnp.int32),
        pltpu.VMEM((C, D), jnp.float32),
        pltpu.VMEM((C, D), jnp.float32),
        pltpu.VMEM((C, D), jnp.float32),
        pltpu.VMEM_SHARED((N, D), jnp.float32),
        pltpu.SemaphoreType.DMA,
        pltpu.SemaphoreType.DMA,
        pltpu.SemaphoreType.DMA,
        pltpu.SemaphoreType.DMA,
        pltpu.SemaphoreType.DMA,
        pltpu.SemaphoreType.DMA,
    ],
)
def _sc_aggregate(h_hbm, ei_hbm, zeros_hbm, out_hbm,
                  src_v, dst_v, buf0, buf1, buf2, agg_sh,
                  sg0, sg1, sg2, ss0, ss1, ss2):
    cid = lax.axis_index("c")
    sid = lax.axis_index("s")
    wid = cid * NS + sid
    bufs = (buf0, buf1, buf2)
    sgs = (sg0, sg1, sg2)
    sss = (ss0, ss1, ss2)

    @pl.when(sid < N // RB)
    def _():
        sl = pl.ds(pl.multiple_of(sid * RB, 8), RB)
        pltpu.sync_copy(zeros_hbm.at[sl], agg_sh.at[sl])

    base = 0
    pltpu.sync_copy(ei_hbm.at[pl.ds(wid * EPW, PHASES[0] * C)],
                    src_v.at[pl.ds(0, PHASES[0] * C)])
    pltpu.sync_copy(ei_hbm.at[pl.ds(E + wid * EPW, PHASES[0] * C)],
                    dst_v.at[pl.ds(0, PHASES[0] * C)])
    plsc.subcore_barrier()

    for n in PHASES:
        _ring3_phase(h_hbm, src_v, dst_v, bufs, sgs, sss, agg_sh, n)
        base += n
        if base < NCH:
            nxt = min(PHB, NCH - base)
            pltpu.sync_copy(ei_hbm.at[pl.ds(wid * EPW + base * C, nxt * C)],
                            src_v.at[pl.ds(0, nxt * C)])
            pltpu.sync_copy(ei_hbm.at[pl.ds(E + wid * EPW + base * C, nxt * C)],
                            dst_v.at[pl.ds(0, nxt * C)])

    plsc.subcore_barrier()

    @pl.when(sid < N // RB)
    def _():
        sl = pl.ds(pl.multiple_of(sid * RB, 8), RB)
        pltpu.sync_copy(agg_sh.at[sl], out_hbm.at[cid, sl])


def _tc_pre_body(feat_ref, w_ref, degp_ref, h_ref):
    d = degp_ref[...]
    deg = d[0, 0, 0, 0, :] + d[1, 0, 0, 0, :]
    norm = lax.rsqrt(jnp.maximum(deg, 1.0))
    t = jnp.dot(feat_ref[...].astype(jnp.bfloat16),
                w_ref[...].astype(jnp.bfloat16),
                preferred_element_type=jnp.float32)
    h_ref[...] = t * norm[:, None]


_tc_pre = pl.pallas_call(
    _tc_pre_body,
    grid=(NP // TB,),
    in_specs=[
        pl.BlockSpec((TB, D), lambda i: (i, 0)),
        pl.BlockSpec((D, D), lambda i: (0, 0)),
        pl.BlockSpec((NC, 2, 1, 1, TB), lambda i: (0, 0, i, 0, 0)),
    ],
    out_specs=pl.BlockSpec((TB, D), lambda i: (i, 0)),
    out_shape=jax.ShapeDtypeStruct((N, D), jnp.float32),
)


def _tc_post_body(parts_ref, degp_ref, b_ref, out_ref):
    p = parts_ref[...]
    d = degp_ref[...]
    deg = d[0, 1, 0, 0, :] + d[1, 1, 0, 0, :]
    norm = lax.rsqrt(jnp.maximum(deg, 1.0))
    agg = (p[0] + p[1]) * norm[:, None]
    out_ref[...] = jnp.maximum(agg + b_ref[...], 0.0)


_tc_post = pl.pallas_call(
    _tc_post_body,
    grid=(NP // TB,),
    in_specs=[
        pl.BlockSpec((NC, TB, D), lambda i: (0, i, 0)),
        pl.BlockSpec((NC, 2, 1, 1, TB), lambda i: (0, 0, i, 0, 0)),
        pl.BlockSpec((1, D), lambda i: (0, 0)),
    ],
    out_specs=pl.BlockSpec((TB, D), lambda i: (i, 0)),
    out_shape=jax.ShapeDtypeStruct((N, D), jnp.float32),
)


def kernel(feat, edge_index, W, b):
    zeros1 = jnp.zeros((NP,), jnp.float32)
    ones_c = jnp.ones((C,), jnp.float32)
    zeros2 = jnp.zeros((N, D), jnp.float32)

    ei_flat = edge_index.reshape(2 * E)
    degp = _sc_degrees(ei_flat, zeros1, ones_c)
    degp5 = degp.reshape(NC, 2, NP // TB, 1, TB)

    h = _tc_pre(feat, W, degp5)
    parts = _sc_aggregate(h, ei_flat, zeros2)
    return _tc_post(parts, degp5, b.reshape(1, D))
